# baseline (speedup 1.0000x reference)
import functools

import jax
import jax.numpy as jnp
from jax import lax
from jax.experimental import pallas as pl
from jax.experimental.pallas import tpu as pltpu
from jax.experimental.pallas import tpu_sc as plsc

M_ROWS = 100000
M_PAD = M_ROWS + 8
D = 256
B_ROWS = 16384
NC, NS = 2, 16
NW = NC * NS
BPW = B_ROWS // NW
CH = 128
NCH = BPW // CH

_mesh = plsc.VectorSubcoreMesh(core_axis_name="c", subcore_axis_name="s")


def _wid():
  return lax.axis_index("s") * NC + lax.axis_index("c")


@functools.partial(
    pl.kernel,
    mesh=_mesh,
    out_type=(jax.ShapeDtypeStruct((B_ROWS, D), jnp.float32),
              jax.ShapeDtypeStruct((B_ROWS, D), jnp.float32)),
    scratch_types=[
        pltpu.VMEM((NCH, CH), jnp.int32),
        pltpu.VMEM((NCH, CH), jnp.int32),
        pltpu.VMEM((NCH, CH), jnp.int32),
        pltpu.VMEM((NCH, CH), jnp.float32),
        pltpu.VMEM((2, CH, D), jnp.float32),
        pltpu.SemaphoreType.DMA,
        pltpu.SemaphoreType.DMA,
        pltpu.SemaphoreType.DMA,
        pltpu.SemaphoreType.DMA,
        pltpu.SemaphoreType.DMA,
    ],
)
def _sc_gather(table_hbm, msgs_hbm, sid_hbm, win_hbm, luid_hbm, ts_hbm, lu_ref,
               h_out, m_out, sid_v, win_v, luid_v, ts_v, rows_v,
               sg0, sg1, sw0, sw1, slu):
  wid = _wid()
  base = wid * BPW
  sg = (sg0, sg1)
  sw = (sw0, sw1)
  pltpu.sync_copy(sid_hbm.at[wid], sid_v)
  pltpu.sync_copy(win_hbm.at[wid], win_v)
  pltpu.sync_copy(luid_hbm.at[wid], luid_v)
  pltpu.sync_copy(ts_hbm.at[wid], ts_v)
  lu_copies = [
      pltpu.async_copy(ts_v.at[ch], lu_ref.at[luid_v.at[ch]], slu)
      for ch in range(NCH)
  ]
  jobs = ([(table_hbm, sid_v, h_out, ch) for ch in range(NCH)]
          + [(msgs_hbm, win_v, m_out, ch) for ch in range(NCH)])
  n = len(jobs)
  gathers = [None] * n
  writes = [None] * n

  def start(j, b):
    src, idx, _, ch = jobs[j]
    return pltpu.async_copy(src.at[idx.at[ch]], rows_v.at[b], sg[b])

  gathers[0] = start(0, 0)
  for j in range(n):
    b = j % 2
    if j + 1 < n:
      nb = (j + 1) % 2
      if writes[j - 1] is not None:
        writes[j - 1].wait()
      gathers[j + 1] = start(j + 1, nb)
    gathers[j].wait()
    _, _, out, ch = jobs[j]
    writes[j] = pltpu.async_copy(
        rows_v.at[b], out.at[pl.ds(base + ch * CH, CH)], sw[b])
  writes[n - 2].wait()
  writes[n - 1].wait()
  for c in lu_copies:
    c.wait()


BLK = 512


def _gru_block(x_ref, h_ref, wih_ref, whh_ref, bih_ref, bhh_ref,
               fcw_ref, fcb_ref, linw_ref, linb_ref, out_ref):
  x = x_ref[...]
  h = h_ref[...]
  gi = jnp.dot(x, wih_ref[...], preferred_element_type=jnp.float32) + bih_ref[...]
  gh = jnp.dot(h, whh_ref[...], preferred_element_type=jnp.float32) + bhh_ref[...]
  r = jax.nn.sigmoid(gi[:, 0:D] + gh[:, 0:D])
  z = jax.nn.sigmoid(gi[:, D:2 * D] + gh[:, D:2 * D])
  n = jnp.tanh(gi[:, 2 * D:3 * D] + r * gh[:, 2 * D:3 * D])
  hy = (1.0 - z) * n + z * h
  pred = jnp.dot(hy, fcw_ref[...], preferred_element_type=jnp.float32) + fcb_ref[...]
  out_ref[...] = jnp.dot(pred, linw_ref[...], preferred_element_type=jnp.float32) + linb_ref[...]


_gru = pl.pallas_call(
    _gru_block,
    grid=(B_ROWS // BLK,),
    in_specs=[
        pl.BlockSpec((BLK, D), lambda i: (i, 0)),
        pl.BlockSpec((BLK, D), lambda i: (i, 0)),
        pl.BlockSpec((D, 3 * D), lambda i: (0, 0)),
        pl.BlockSpec((D, 3 * D), lambda i: (0, 0)),
        pl.BlockSpec((1, 3 * D), lambda i: (0, 0)),
        pl.BlockSpec((1, 3 * D), lambda i: (0, 0)),
        pl.BlockSpec((D, 64), lambda i: (0, 0)),
        pl.BlockSpec((1, 64), lambda i: (0, 0)),
        pl.BlockSpec((64, D), lambda i: (0, 0)),
        pl.BlockSpec((1, D), lambda i: (0, 0)),
    ],
    out_specs=pl.BlockSpec((BLK, D), lambda i: (i, 0)),
    out_shape=jax.ShapeDtypeStruct((B_ROWS, D), jnp.float32),
)


@functools.partial(
    pl.kernel,
    mesh=_mesh,
    out_type=(),
    scratch_types=[
        pltpu.VMEM((NCH, CH), jnp.int32),
        pltpu.VMEM((2, CH, D), jnp.float32),
        pltpu.SemaphoreType.DMA,
        pltpu.SemaphoreType.DMA,
        pltpu.SemaphoreType.DMA,
        pltpu.SemaphoreType.DMA,
    ],
)
def _sc_scatter(newmem_hbm, ids_hbm, table_ref, ids_v, rows_v, sl0, sl1, ss0, ss1):
  wid = _wid()
  base = wid * BPW
  sl = (sl0, sl1)
  ss = (ss0, ss1)
  pltpu.sync_copy(ids_hbm.at[wid], ids_v)
  loads = [None] * NCH
  scatters = [None] * NCH
  loads[0] = pltpu.async_copy(newmem_hbm.at[pl.ds(base, CH)], rows_v.at[0], sl[0])
  for ch in range(NCH):
    b = ch % 2
    if ch + 1 < NCH:
      nb = (ch + 1) % 2
      if scatters[ch - 1] is not None:
        scatters[ch - 1].wait()
      loads[ch + 1] = pltpu.async_copy(
          newmem_hbm.at[pl.ds(base + (ch + 1) * CH, CH)], rows_v.at[nb], sl[nb])
    loads[ch].wait()
    scatters[ch] = pltpu.async_copy(rows_v.at[b], table_ref.at[ids_v.at[ch]], ss[b])
  scatters[NCH - 2].wait()
  scatters[NCH - 1].wait()


def kernel(memory_table, last_update, unique_node_ids, unique_messages,
           timestamps, w_ih, w_hh, b_ih, b_hh, fc_w, fc_b, lin_w, lin_b):
  ids = unique_node_ids

  table_ref = jax.new_ref(memory_table)
  lu_ref = jax.new_ref(jnp.pad(last_update, (0, M_PAD - M_ROWS)))

  iota = jnp.arange(B_ROWS, dtype=jnp.int32)
  key, ts_sorted = lax.sort(((ids << 14) | iota, timestamps), num_keys=1)
  sid = key >> 14
  sp = key & (B_ROWS - 1)
  boundary = jnp.concatenate([sid[1:] != sid[:-1], jnp.ones((1,), jnp.bool_)])
  enc = jnp.where(boundary, (iota << 14) | sp, jnp.int32(1 << 30))
  winner = lax.cummin(enc, axis=0, reverse=True) & (B_ROWS - 1)
  lu_ids = jnp.where(boundary, sid, jnp.int32(M_ROWS))

  rs = lambda a: a.reshape(NW, NCH, CH)
  h_s, msgs_s = _sc_gather(memory_table, unique_messages, rs(sid), rs(winner),
                           rs(lu_ids), rs(ts_sorted), lu_ref)

  new_mem = _gru(
      msgs_s, h_s,
      w_ih.T, w_hh.T,
      b_ih.reshape(1, 3 * D), b_hh.reshape(1, 3 * D),
      fc_w.T, fc_b.reshape(1, 64),
      lin_w.T, lin_b.reshape(1, D),
  )

  _sc_scatter(new_mem, rs(sid), table_ref)
  return jax.freeze(table_ref), jax.freeze(lu_ref)[:M_ROWS]


# speedup vs baseline: 1.6421x; 1.6421x over previous
import functools

import jax
import jax.numpy as jnp
from jax import lax
from jax.experimental import pallas as pl
from jax.experimental.pallas import tpu as pltpu
from jax.experimental.pallas import tpu_sc as plsc

M_ROWS = 100000
M_PAD = M_ROWS + 8
D = 256
B_ROWS = 16384
NC, NS = 2, 16
NW = NC * NS
BPW = B_ROWS // NW
CH = 128
NCH = BPW // CH

_mesh = plsc.VectorSubcoreMesh(core_axis_name="c", subcore_axis_name="s")


def _wid():
  return lax.axis_index("s") * NC + lax.axis_index("c")


@functools.partial(
    pl.kernel,
    mesh=_mesh,
    out_type=(jax.ShapeDtypeStruct((B_ROWS, D), jnp.float32),
              jax.ShapeDtypeStruct((B_ROWS, D), jnp.float32)),
    scratch_types=[
        pltpu.VMEM((NCH, CH), jnp.int32),
        pltpu.VMEM((NCH, CH), jnp.int32),
        pltpu.VMEM((NCH, CH), jnp.int32),
        pltpu.VMEM((NCH, CH), jnp.float32),
        pltpu.VMEM((2, CH, D), jnp.float32),
        pltpu.SemaphoreType.DMA,
        pltpu.SemaphoreType.DMA,
        pltpu.SemaphoreType.DMA,
        pltpu.SemaphoreType.DMA,
        pltpu.SemaphoreType.DMA,
    ],
)
def _sc_gather(table_hbm, msgs_hbm, sid_hbm, win_hbm, luid_hbm, ts_hbm, lu_ref,
               h_out, m_out, sid_v, win_v, luid_v, ts_v, rows_v,
               sg0, sg1, sw0, sw1, slu):
  wid = _wid()
  base = wid * BPW
  sg = (sg0, sg1)
  sw = (sw0, sw1)
  pltpu.sync_copy(sid_hbm.at[wid], sid_v)
  pltpu.sync_copy(win_hbm.at[wid], win_v)
  pltpu.sync_copy(luid_hbm.at[wid], luid_v)
  pltpu.sync_copy(ts_hbm.at[wid], ts_v)
  lu_copies = [
      pltpu.async_copy(ts_v.at[ch], lu_ref.at[luid_v.at[ch]], slu)
      for ch in range(NCH)
  ]
  jobs = ([(table_hbm, sid_v, h_out, ch) for ch in range(NCH)]
          + [(msgs_hbm, win_v, m_out, ch) for ch in range(NCH)])
  n = len(jobs)
  gathers = [None] * n
  writes = [None] * n

  def start(j, b):
    src, idx, _, ch = jobs[j]
    return pltpu.async_copy(src.at[idx.at[ch]], rows_v.at[b], sg[b])

  gathers[0] = start(0, 0)
  for j in range(n):
    b = j % 2
    if j + 1 < n:
      nb = (j + 1) % 2
      if writes[j - 1] is not None:
        writes[j - 1].wait()
      gathers[j + 1] = start(j + 1, nb)
    gathers[j].wait()
    _, _, out, ch = jobs[j]
    writes[j] = pltpu.async_copy(
        rows_v.at[b], out.at[pl.ds(base + ch * CH, CH)], sw[b])
  writes[n - 2].wait()
  writes[n - 1].wait()
  for c in lu_copies:
    c.wait()


BLK = 512


def _gru_block(x_ref, h_ref, wih_ref, whh_ref, bih_ref, bhh_ref,
               fcw_ref, fcb_ref, linw_ref, linb_ref, out_ref):
  x = x_ref[...]
  h = h_ref[...]
  gi = jnp.dot(x, wih_ref[...], preferred_element_type=jnp.float32) + bih_ref[...]
  gh = jnp.dot(h, whh_ref[...], preferred_element_type=jnp.float32) + bhh_ref[...]
  r = jax.nn.sigmoid(gi[:, 0:D] + gh[:, 0:D])
  z = jax.nn.sigmoid(gi[:, D:2 * D] + gh[:, D:2 * D])
  n = jnp.tanh(gi[:, 2 * D:3 * D] + r * gh[:, 2 * D:3 * D])
  hy = (1.0 - z) * n + z * h
  pred = jnp.dot(hy, fcw_ref[...], preferred_element_type=jnp.float32) + fcb_ref[...]
  out_ref[...] = jnp.dot(pred, linw_ref[...], preferred_element_type=jnp.float32) + linb_ref[...]


_gru = pl.pallas_call(
    _gru_block,
    grid=(B_ROWS // BLK,),
    in_specs=[
        pl.BlockSpec((BLK, D), lambda i: (i, 0)),
        pl.BlockSpec((BLK, D), lambda i: (i, 0)),
        pl.BlockSpec((D, 3 * D), lambda i: (0, 0)),
        pl.BlockSpec((D, 3 * D), lambda i: (0, 0)),
        pl.BlockSpec((1, 3 * D), lambda i: (0, 0)),
        pl.BlockSpec((1, 3 * D), lambda i: (0, 0)),
        pl.BlockSpec((D, 64), lambda i: (0, 0)),
        pl.BlockSpec((1, 64), lambda i: (0, 0)),
        pl.BlockSpec((64, D), lambda i: (0, 0)),
        pl.BlockSpec((1, D), lambda i: (0, 0)),
    ],
    out_specs=pl.BlockSpec((BLK, D), lambda i: (i, 0)),
    out_shape=jax.ShapeDtypeStruct((B_ROWS, D), jnp.float32),
)


@functools.partial(
    pl.kernel,
    mesh=_mesh,
    out_type=(),
    scratch_types=[
        pltpu.VMEM((NCH, CH), jnp.int32),
        pltpu.VMEM((2, CH, D), jnp.float32),
        pltpu.SemaphoreType.DMA,
        pltpu.SemaphoreType.DMA,
        pltpu.SemaphoreType.DMA,
        pltpu.SemaphoreType.DMA,
    ],
)
def _sc_scatter(newmem_hbm, ids_hbm, table_ref, ids_v, rows_v, sl0, sl1, ss0, ss1):
  wid = _wid()
  base = wid * BPW
  sl = (sl0, sl1)
  ss = (ss0, ss1)
  pltpu.sync_copy(ids_hbm.at[wid], ids_v)
  loads = [None] * NCH
  scatters = [None] * NCH
  loads[0] = pltpu.async_copy(newmem_hbm.at[pl.ds(base, CH)], rows_v.at[0], sl[0])
  for ch in range(NCH):
    b = ch % 2
    if ch + 1 < NCH:
      nb = (ch + 1) % 2
      if scatters[ch - 1] is not None:
        scatters[ch - 1].wait()
      loads[ch + 1] = pltpu.async_copy(
          newmem_hbm.at[pl.ds(base + (ch + 1) * CH, CH)], rows_v.at[nb], sl[nb])
    loads[ch].wait()
    scatters[ch] = pltpu.async_copy(rows_v.at[b], table_ref.at[ids_v.at[ch]], ss[b])
  scatters[NCH - 2].wait()
  scatters[NCH - 1].wait()


def kernel(memory_table, last_update, unique_node_ids, unique_messages,
           timestamps, w_ih, w_hh, b_ih, b_hh, fc_w, fc_b, lin_w, lin_b):
  ids = unique_node_ids

  table_ref = jax.new_ref(memory_table)
  lu_ref = jax.new_ref(last_update)

  iota = jnp.arange(B_ROWS, dtype=jnp.int32)
  key, ts_sorted = lax.sort(((ids << 14) | iota, timestamps), num_keys=1)
  sid = key >> 14
  sp = key & (B_ROWS - 1)
  boundary = jnp.concatenate([sid[1:] != sid[:-1], jnp.ones((1,), jnp.bool_)])
  enc = jnp.where(boundary, (iota << 14) | sp, jnp.int32(1 << 30))
  nbenc = lax.cummin(enc, axis=0, reverse=True)
  winner = nbenc & (B_ROWS - 1)
  tsw_sorted = jnp.take(ts_sorted, nbenc >> 14)

  rs = lambda a: a.reshape(NW, NCH, CH)
  h_s, msgs_s = _sc_gather(memory_table, unique_messages, rs(sid), rs(winner),
                           rs(sid), rs(tsw_sorted), lu_ref)

  new_mem = _gru(
      msgs_s, h_s,
      w_ih.T, w_hh.T,
      b_ih.reshape(1, 3 * D), b_hh.reshape(1, 3 * D),
      fc_w.T, fc_b.reshape(1, 64),
      lin_w.T, lin_b.reshape(1, D),
  )

  _sc_scatter(new_mem, rs(sid), table_ref)
  return jax.freeze(table_ref), jax.freeze(lu_ref)


# speedup vs baseline: 1.6513x; 1.0056x over previous
import functools

import jax
import jax.numpy as jnp
from jax import lax
from jax.experimental import pallas as pl
from jax.experimental.pallas import tpu as pltpu
from jax.experimental.pallas import tpu_sc as plsc

M_ROWS = 100000
M_PAD = M_ROWS + 8
D = 256
B_ROWS = 16384
NC, NS = 2, 16
NW = NC * NS
BPW = B_ROWS // NW
CH = 128
NCH = BPW // CH

_mesh = plsc.VectorSubcoreMesh(core_axis_name="c", subcore_axis_name="s")


def _wid():
  return lax.axis_index("s") * NC + lax.axis_index("c")


@functools.partial(
    pl.kernel,
    mesh=_mesh,
    out_type=(jax.ShapeDtypeStruct((B_ROWS, D), jnp.float32),
              jax.ShapeDtypeStruct((B_ROWS, D), jnp.float32)),
    scratch_types=[
        pltpu.VMEM((NCH, CH), jnp.int32),
        pltpu.VMEM((NCH, CH), jnp.int32),
        pltpu.VMEM((NCH, CH), jnp.float32),
        pltpu.VMEM((2, CH, D), jnp.float32),
        pltpu.SemaphoreType.DMA,
        pltpu.SemaphoreType.DMA,
        pltpu.SemaphoreType.DMA,
        pltpu.SemaphoreType.DMA,
        pltpu.SemaphoreType.DMA,
    ],
)
def _sc_gather(table_hbm, msgs_hbm, sid_hbm, win_hbm, ts_hbm, lu_ref,
               h_out, m_out, sid_v, win_v, ts_v, rows_v,
               sg0, sg1, sw0, sw1, slu):
  wid = _wid()
  base = wid * BPW
  sg = (sg0, sg1)
  sw = (sw0, sw1)
  pltpu.sync_copy(sid_hbm.at[wid], sid_v)
  pltpu.sync_copy(win_hbm.at[wid], win_v)
  pltpu.sync_copy(ts_hbm.at[wid], ts_v)
  lu_copies = [
      pltpu.async_copy(ts_v.at[ch], lu_ref.at[sid_v.at[ch]], slu)
      for ch in range(NCH)
  ]
  jobs = ([(table_hbm, sid_v, h_out, ch) for ch in range(NCH)]
          + [(msgs_hbm, win_v, m_out, ch) for ch in range(NCH)])
  n = len(jobs)
  gathers = [None] * n
  writes = [None] * n

  def start(j, b):
    src, idx, _, ch = jobs[j]
    return pltpu.async_copy(src.at[idx.at[ch]], rows_v.at[b], sg[b])

  gathers[0] = start(0, 0)
  for j in range(n):
    b = j % 2
    if j + 1 < n:
      nb = (j + 1) % 2
      if writes[j - 1] is not None:
        writes[j - 1].wait()
      gathers[j + 1] = start(j + 1, nb)
    gathers[j].wait()
    _, _, out, ch = jobs[j]
    writes[j] = pltpu.async_copy(
        rows_v.at[b], out.at[pl.ds(base + ch * CH, CH)], sw[b])
  writes[n - 2].wait()
  writes[n - 1].wait()
  for c in lu_copies:
    c.wait()


BLK = 512


def _gru_block(x_ref, h_ref, wih_ref, whh_ref, bih_ref, bhh_ref,
               fcw_ref, fcb_ref, linw_ref, linb_ref, out_ref):
  x = x_ref[...]
  h = h_ref[...]
  gi = jnp.dot(x, wih_ref[...], preferred_element_type=jnp.float32) + bih_ref[...]
  gh = jnp.dot(h, whh_ref[...], preferred_element_type=jnp.float32) + bhh_ref[...]
  r = jax.nn.sigmoid(gi[:, 0:D] + gh[:, 0:D])
  z = jax.nn.sigmoid(gi[:, D:2 * D] + gh[:, D:2 * D])
  n = jnp.tanh(gi[:, 2 * D:3 * D] + r * gh[:, 2 * D:3 * D])
  hy = (1.0 - z) * n + z * h
  pred = jnp.dot(hy, fcw_ref[...], preferred_element_type=jnp.float32) + fcb_ref[...]
  out_ref[...] = jnp.dot(pred, linw_ref[...], preferred_element_type=jnp.float32) + linb_ref[...]


_gru = pl.pallas_call(
    _gru_block,
    grid=(B_ROWS // BLK,),
    in_specs=[
        pl.BlockSpec((BLK, D), lambda i: (i, 0)),
        pl.BlockSpec((BLK, D), lambda i: (i, 0)),
        pl.BlockSpec((D, 3 * D), lambda i: (0, 0)),
        pl.BlockSpec((D, 3 * D), lambda i: (0, 0)),
        pl.BlockSpec((1, 3 * D), lambda i: (0, 0)),
        pl.BlockSpec((1, 3 * D), lambda i: (0, 0)),
        pl.BlockSpec((D, 64), lambda i: (0, 0)),
        pl.BlockSpec((1, 64), lambda i: (0, 0)),
        pl.BlockSpec((64, D), lambda i: (0, 0)),
        pl.BlockSpec((1, D), lambda i: (0, 0)),
    ],
    out_specs=pl.BlockSpec((BLK, D), lambda i: (i, 0)),
    out_shape=jax.ShapeDtypeStruct((B_ROWS, D), jnp.float32),
)


@functools.partial(
    pl.kernel,
    mesh=_mesh,
    out_type=(),
    scratch_types=[
        pltpu.VMEM((NCH, CH), jnp.int32),
        pltpu.VMEM((2, CH, D), jnp.float32),
        pltpu.SemaphoreType.DMA,
        pltpu.SemaphoreType.DMA,
        pltpu.SemaphoreType.DMA,
        pltpu.SemaphoreType.DMA,
    ],
)
def _sc_scatter(newmem_hbm, ids_hbm, table_ref, ids_v, rows_v, sl0, sl1, ss0, ss1):
  wid = _wid()
  base = wid * BPW
  sl = (sl0, sl1)
  ss = (ss0, ss1)
  pltpu.sync_copy(ids_hbm.at[wid], ids_v)
  loads = [None] * NCH
  scatters = [None] * NCH
  loads[0] = pltpu.async_copy(newmem_hbm.at[pl.ds(base, CH)], rows_v.at[0], sl[0])
  for ch in range(NCH):
    b = ch % 2
    if ch + 1 < NCH:
      nb = (ch + 1) % 2
      if scatters[ch - 1] is not None:
        scatters[ch - 1].wait()
      loads[ch + 1] = pltpu.async_copy(
          newmem_hbm.at[pl.ds(base + (ch + 1) * CH, CH)], rows_v.at[nb], sl[nb])
    loads[ch].wait()
    scatters[ch] = pltpu.async_copy(rows_v.at[b], table_ref.at[ids_v.at[ch]], ss[b])
  scatters[NCH - 2].wait()
  scatters[NCH - 1].wait()


def kernel(memory_table, last_update, unique_node_ids, unique_messages,
           timestamps, w_ih, w_hh, b_ih, b_hh, fc_w, fc_b, lin_w, lin_b):
  ids = unique_node_ids

  iota = jnp.arange(B_ROWS, dtype=jnp.int32)
  key, ts_sorted = lax.sort(((ids << 14) | iota, timestamps), num_keys=1)

  mem_b = lax.optimization_barrier((memory_table, key))[0]
  table_ref = jax.new_ref(mem_b)
  lu_ref = jax.new_ref(last_update)

  sid = key >> 14
  sp = key & (B_ROWS - 1)
  boundary = jnp.concatenate([sid[1:] != sid[:-1], jnp.ones((1,), jnp.bool_)])
  enc = jnp.where(boundary, (iota << 14) | sp, jnp.int32(1 << 30))
  nbenc = lax.cummin(enc, axis=0, reverse=True)
  winner = nbenc & (B_ROWS - 1)
  tsw_sorted = jnp.take(ts_sorted, nbenc >> 14)

  rs = lambda a: a.reshape(NW, NCH, CH)
  h_s, msgs_s = _sc_gather(memory_table, unique_messages, rs(sid), rs(winner),
                           rs(tsw_sorted), lu_ref)

  new_mem = _gru(
      msgs_s, h_s,
      w_ih.T, w_hh.T,
      b_ih.reshape(1, 3 * D), b_hh.reshape(1, 3 * D),
      fc_w.T, fc_b.reshape(1, 64),
      lin_w.T, lin_b.reshape(1, D),
  )

  _sc_scatter(new_mem, rs(sid), table_ref)
  return jax.freeze(table_ref), jax.freeze(lu_ref)
